# baseline (speedup 1.0000x reference)
import functools

import jax
import jax.numpy as jnp
from jax import lax
from jax.experimental import pallas as pl
from jax.experimental.pallas import tpu as pltpu
from jax.experimental.pallas import tpu_sc as plsc

BATCH = 16384
EMB = 64

_NC = 2
_NS = 16
_NW = _NC * _NS

_TBLK = 2048


def _detrans_body(t_ref, o_ref):
    y = t_ref[...].T
    h = _TBLK // 2
    o_ref[...] = jnp.concatenate([y[:h], y[h:]], axis=1)


def _detranspose(tT):
    n = tT.shape[1]
    grid = (pl.cdiv(n, _TBLK),)
    in_spec = pl.BlockSpec((EMB, _TBLK), lambda i: (0, i))
    out_spec = pl.BlockSpec((_TBLK // 2, 2 * EMB), lambda i: (i, 0))
    nout = (_TBLK // 2) * pl.cdiv(n, _TBLK)
    out_shape = jax.ShapeDtypeStruct((nout, 2 * EMB), jnp.float32)
    return pl.pallas_call(
        _detrans_body,
        grid=grid,
        in_specs=[in_spec],
        out_specs=out_spec,
        out_shape=out_shape,
    )(tT)


_CH = 256


def _gather_body(uids_hbm, iids_hbm, t1_hbm, t2_hbm, x_hbm,
                 uidx_v, iidx_v, uhalf_v, upar_v,
                 urows_v, irows_v, cat_v, sem_u, sem_i, bpw):
    wid = lax.axis_index("s") * _NC + lax.axis_index("c")
    base = wid * bpw
    pltpu.sync_copy(uids_hbm.at[pl.ds(base, bpw)], uidx_v)
    pltpu.sync_copy(iids_hbm.at[pl.ds(base, bpw)], iidx_v)
    for j in range(bpw // 16):
        s = pl.ds(16 * j, 16)
        u = uidx_v[s]
        uhalf_v[s] = ((u >> 11) << 10) | (u & 1023)
        upar_v[s] = ((u >> 10) & 1) << 6

    for c in range(bpw // _CH):
        off = c * _CH
        cu = pltpu.async_copy(t1_hbm.at[uhalf_v.at[pl.ds(off, _CH)]],
                              urows_v, sem_u)
        ci = pltpu.async_copy(t2_hbm.at[iidx_v.at[pl.ds(off, _CH)]],
                              irows_v, sem_i)
        cu.wait()
        ci.wait()

        def repack(j, carry):
            rows = lax.broadcasted_iota(jnp.int32, (16,), 0) + 16 * j
            pu = upar_v[pl.ds(off + 16 * j, 16)]
            for col in range(EMB):
                cc = jnp.full((16,), col, jnp.int32)
                vu = plsc.load_gather(urows_v, [rows, pu + col])
                plsc.store_scatter(cat_v, [rows, cc], vu)
                vi = plsc.load_gather(irows_v, [rows, cc])
                plsc.store_scatter(cat_v, [rows, cc + EMB], vi)
            return carry

        lax.fori_loop(0, _CH // 16, repack, 0)
        pltpu.sync_copy(cat_v, x_hbm.at[pl.ds(base + off, _CH)])


def _sc_gather(user_ids, item_ids, t1, t2):
    bpw = BATCH // _NW
    mesh = plsc.VectorSubcoreMesh(core_axis_name="c", subcore_axis_name="s")
    out_type = jax.ShapeDtypeStruct((BATCH, 2 * EMB), jnp.float32)
    scratch = [
        pltpu.VMEM((bpw,), jnp.int32),
        pltpu.VMEM((bpw,), jnp.int32),
        pltpu.VMEM((bpw,), jnp.int32),
        pltpu.VMEM((bpw,), jnp.int32),
        pltpu.VMEM((_CH, 2 * EMB), jnp.float32),
        pltpu.VMEM((_CH, EMB), jnp.float32),
        pltpu.VMEM((_CH, 2 * EMB), jnp.float32),
        pltpu.SemaphoreType.DMA,
        pltpu.SemaphoreType.DMA,
    ]
    k = pl.kernel(
        functools.partial(_gather_body, bpw=bpw),
        out_type=out_type,
        mesh=mesh,
        scratch_types=scratch,
        compiler_params=pltpu.CompilerParams(
            use_tc_tiling_on_sc=False,
            needs_layout_passes=False,
        ),
    )
    return k(user_ids, item_ids, t1, t2)


def _layernorm(x, g, b, eps=1e-5):
    mu = jnp.mean(x, axis=-1, keepdims=True)
    var = jnp.mean((x - mu) ** 2, axis=-1, keepdims=True)
    return (x - mu) / jnp.sqrt(var + eps) * g + b


def _gelu(x):
    return 0.5 * x * (1.0 + lax.erf(x * (2.0 ** -0.5)))


def _mlp_body(xin, W0, b0, g0, beta0, W1, b1, g1, beta1,
              W2, b2, g2, beta2, W_out, b_out, out):
    dot = functools.partial(jnp.dot, preferred_element_type=jnp.float32)
    x = dot(xin[...], W0[...]) + b0[...]
    x = _gelu(_layernorm(x, g0[...], beta0[...]))
    x = dot(x, W1[...]) + b1[...]
    x = _gelu(_layernorm(x, g1[...], beta1[...]))
    x = dot(x, W2[...]) + b2[...]
    x = _gelu(_layernorm(x, g2[...], beta2[...]))
    out[...] = dot(x, W_out[...]) + b_out[...]


def _tc_mlp(x, W0, b0, g0, beta0, W1, b1, g1, beta1,
            W2, b2, g2, beta2, W_out, b_out):
    blk = 2048
    grid = (BATCH // blk,)

    def full_spec(a):
        return pl.BlockSpec(a.shape, lambda i: (0,) * a.ndim)

    b0r, g0r, beta0r = (a.reshape(1, -1) for a in (b0, g0, beta0))
    b1r, g1r, beta1r = (a.reshape(1, -1) for a in (b1, g1, beta1))
    b2r, g2r, beta2r = (a.reshape(1, -1) for a in (b2, g2, beta2))
    b_outr = b_out.reshape(1, -1)

    args = (x, W0, b0r, g0r, beta0r, W1, b1r, g1r, beta1r,
            W2, b2r, g2r, beta2r, W_out, b_outr)
    in_specs = ([pl.BlockSpec((blk, 2 * EMB), lambda i: (i, 0))]
                + [full_spec(a) for a in args[1:]])
    return pl.pallas_call(
        _mlp_body,
        grid=grid,
        in_specs=in_specs,
        out_specs=pl.BlockSpec((blk, 1), lambda i: (i, 0)),
        out_shape=jax.ShapeDtypeStruct((BATCH, 1), jnp.float32),
    )(*args)


def kernel(user_ids, item_ids, user_table, item_table,
           W0, b0, g0, beta0, W1, b1, g1, beta1, W2, b2, g2, beta2,
           W_out, b_out):
    t1 = _detranspose(user_table.T)
    x = _sc_gather(user_ids.astype(jnp.int32), item_ids.astype(jnp.int32),
                   t1, item_table)
    return _tc_mlp(x, W0, b0, g0, beta0, W1, b1, g1, beta1,
                   W2, b2, g2, beta2, W_out, b_out)


# speedup vs baseline: 1.8726x; 1.8726x over previous
import functools

import jax
import jax.numpy as jnp
from jax import lax
from jax.experimental import pallas as pl
from jax.experimental.pallas import tpu as pltpu
from jax.experimental.pallas import tpu_sc as plsc

BATCH = 16384
EMB = 64

_NC = 2
_NS = 16
_NW = _NC * _NS

_TBLK = 8192


def _detrans_body(t1_ref, t2_ref, o1_ref, o2_ref):
    h = _TBLK // 2
    for t_ref, o_ref in ((t1_ref, o1_ref), (t2_ref, o2_ref)):
        y = t_ref[...].T
        o_ref[...] = jnp.concatenate([y[:h], y[h:]], axis=1)


def _detranspose(t1T, t2T):
    n = t1T.shape[1]
    grid = (pl.cdiv(n, _TBLK),)
    in_spec = pl.BlockSpec((EMB, _TBLK), lambda i: (0, i))
    out_spec = pl.BlockSpec((_TBLK // 2, 2 * EMB), lambda i: (i, 0))
    nout = (_TBLK // 2) * pl.cdiv(n, _TBLK)
    out_shape = jax.ShapeDtypeStruct((nout, 2 * EMB), jnp.float32)
    return pl.pallas_call(
        _detrans_body,
        grid=grid,
        in_specs=[in_spec, in_spec],
        out_specs=[out_spec, out_spec],
        out_shape=[out_shape, out_shape],
    )(t1T, t2T)


_CH = 256


def _gather_body(uids_hbm, iids_hbm, t1_hbm, t2_hbm, x_hbm,
                 uidx_v, iidx_v, uhalf_v, ihalf_v, upar_v, ipar_v,
                 urows_v, irows_v, cat_v, sem_u, sem_i, bpw):
    wid = lax.axis_index("s") * _NC + lax.axis_index("c")
    base = wid * bpw
    pltpu.sync_copy(uids_hbm.at[pl.ds(base, bpw)], uidx_v)
    pltpu.sync_copy(iids_hbm.at[pl.ds(base, bpw)], iidx_v)
    hb = _TBLK // 2
    hs = hb.bit_length() - 1
    for j in range(bpw // 16):
        s = pl.ds(16 * j, 16)
        u = uidx_v[s]
        i = iidx_v[s]
        uhalf_v[s] = ((u >> (hs + 1)) << hs) | (u & (hb - 1))
        ihalf_v[s] = ((i >> (hs + 1)) << hs) | (i & (hb - 1))
        upar_v[s] = ((u >> hs) & 1) << 6
        ipar_v[s] = ((i >> hs) & 1) << 6

    for c in range(bpw // _CH):
        off = c * _CH
        cu = pltpu.async_copy(t1_hbm.at[uhalf_v.at[pl.ds(off, _CH)]],
                              urows_v, sem_u)
        ci = pltpu.async_copy(t2_hbm.at[ihalf_v.at[pl.ds(off, _CH)]],
                              irows_v, sem_i)
        cu.wait()
        ci.wait()

        def repack(j, carry):
            rows = lax.broadcasted_iota(jnp.int32, (16,), 0) + 16 * j
            pu = upar_v[pl.ds(off + 16 * j, 16)]
            pi = ipar_v[pl.ds(off + 16 * j, 16)]
            for col in range(EMB):
                cc = jnp.full((16,), col, jnp.int32)
                vu = plsc.load_gather(urows_v, [rows, pu + col])
                plsc.store_scatter(cat_v, [rows, cc], vu)
                vi = plsc.load_gather(irows_v, [rows, pi + col])
                plsc.store_scatter(cat_v, [rows, cc + EMB], vi)
            return carry

        lax.fori_loop(0, _CH // 16, repack, 0)
        pltpu.sync_copy(cat_v, x_hbm.at[pl.ds(base + off, _CH)])


def _sc_gather(user_ids, item_ids, t1, t2):
    bpw = BATCH // _NW
    mesh = plsc.VectorSubcoreMesh(core_axis_name="c", subcore_axis_name="s")
    out_type = jax.ShapeDtypeStruct((BATCH, 2 * EMB), jnp.float32)
    scratch = [
        pltpu.VMEM((bpw,), jnp.int32),
        pltpu.VMEM((bpw,), jnp.int32),
        pltpu.VMEM((bpw,), jnp.int32),
        pltpu.VMEM((bpw,), jnp.int32),
        pltpu.VMEM((bpw,), jnp.int32),
        pltpu.VMEM((bpw,), jnp.int32),
        pltpu.VMEM((_CH, 2 * EMB), jnp.float32),
        pltpu.VMEM((_CH, 2 * EMB), jnp.float32),
        pltpu.VMEM((_CH, 2 * EMB), jnp.float32),
        pltpu.SemaphoreType.DMA,
        pltpu.SemaphoreType.DMA,
    ]
    k = pl.kernel(
        functools.partial(_gather_body, bpw=bpw),
        out_type=out_type,
        mesh=mesh,
        scratch_types=scratch,
        compiler_params=pltpu.CompilerParams(
            use_tc_tiling_on_sc=False,
            needs_layout_passes=False,
        ),
    )
    return k(user_ids, item_ids, t1, t2)


def _layernorm(x, g, b, eps=1e-5):
    mu = jnp.mean(x, axis=-1, keepdims=True)
    var = jnp.mean((x - mu) ** 2, axis=-1, keepdims=True)
    return (x - mu) / jnp.sqrt(var + eps) * g + b


def _gelu(x):
    return 0.5 * x * (1.0 + lax.erf(x * (2.0 ** -0.5)))


def _mlp_body(xin, W0, b0, g0, beta0, W1, b1, g1, beta1,
              W2, b2, g2, beta2, W_out, b_out, out):
    dot = functools.partial(jnp.dot, preferred_element_type=jnp.float32)
    x = dot(xin[...], W0[...]) + b0[...]
    x = _gelu(_layernorm(x, g0[...], beta0[...]))
    x = dot(x, W1[...]) + b1[...]
    x = _gelu(_layernorm(x, g1[...], beta1[...]))
    x = dot(x, W2[...]) + b2[...]
    x = _gelu(_layernorm(x, g2[...], beta2[...]))
    out[...] = dot(x, W_out[...]) + b_out[...]


def _tc_mlp(x, W0, b0, g0, beta0, W1, b1, g1, beta1,
            W2, b2, g2, beta2, W_out, b_out):
    blk = 2048
    grid = (BATCH // blk,)

    def full_spec(a):
        return pl.BlockSpec(a.shape, lambda i: (0,) * a.ndim)

    b0r, g0r, beta0r = (a.reshape(1, -1) for a in (b0, g0, beta0))
    b1r, g1r, beta1r = (a.reshape(1, -1) for a in (b1, g1, beta1))
    b2r, g2r, beta2r = (a.reshape(1, -1) for a in (b2, g2, beta2))
    b_outr = b_out.reshape(1, -1)

    args = (x, W0, b0r, g0r, beta0r, W1, b1r, g1r, beta1r,
            W2, b2r, g2r, beta2r, W_out, b_outr)
    in_specs = ([pl.BlockSpec((blk, 2 * EMB), lambda i: (i, 0))]
                + [full_spec(a) for a in args[1:]])
    return pl.pallas_call(
        _mlp_body,
        grid=grid,
        in_specs=in_specs,
        out_specs=pl.BlockSpec((blk, 1), lambda i: (i, 0)),
        out_shape=jax.ShapeDtypeStruct((BATCH, 1), jnp.float32),
    )(*args)


def kernel(user_ids, item_ids, user_table, item_table,
           W0, b0, g0, beta0, W1, b1, g1, beta1, W2, b2, g2, beta2,
           W_out, b_out):
    t1, t2 = _detranspose(user_table.T, item_table.T)
    x = _sc_gather(user_ids.astype(jnp.int32), item_ids.astype(jnp.int32),
                   t1, t2)
    return _tc_mlp(x, W0, b0, g0, beta0, W1, b1, g1, beta1,
                   W2, b2, g2, beta2, W_out, b_out)


# speedup vs baseline: 1.8889x; 1.0087x over previous
import functools

import jax
import jax.numpy as jnp
from jax import lax
from jax.experimental import pallas as pl
from jax.experimental.pallas import tpu as pltpu
from jax.experimental.pallas import tpu_sc as plsc

BATCH = 16384
EMB = 64

_NC = 2
_NS = 16
_NW = _NC * _NS

_TBLK = 16384


def _detrans_body(t1_ref, t2_ref, o1_ref, o2_ref):
    h = _TBLK // 2
    for t_ref, o_ref in ((t1_ref, o1_ref), (t2_ref, o2_ref)):
        y = t_ref[...].T
        o_ref[:, :EMB] = y[:h]
        o_ref[:, EMB:] = y[h:]


def _detranspose(t1T, t2T):
    n = t1T.shape[1]
    grid = (pl.cdiv(n, _TBLK),)
    in_spec = pl.BlockSpec((EMB, _TBLK), lambda i: (0, i))
    out_spec = pl.BlockSpec((_TBLK // 2, 2 * EMB), lambda i: (i, 0))
    nout = (_TBLK // 2) * pl.cdiv(n, _TBLK)
    out_shape = jax.ShapeDtypeStruct((nout, 2 * EMB), jnp.float32)
    return pl.pallas_call(
        _detrans_body,
        grid=grid,
        in_specs=[in_spec, in_spec],
        out_specs=[out_spec, out_spec],
        out_shape=[out_shape, out_shape],
    )(t1T, t2T)


_CH = 256


def _gather_body(uids_hbm, iids_hbm, t1_hbm, t2_hbm, x_hbm,
                 uidx_v, iidx_v, uhalf_v, ihalf_v, upar_v, ipar_v,
                 urows_v, irows_v, cat_v, sem_u, sem_i, bpw):
    wid = lax.axis_index("s") * _NC + lax.axis_index("c")
    base = wid * bpw
    pltpu.sync_copy(uids_hbm.at[pl.ds(base, bpw)], uidx_v)
    pltpu.sync_copy(iids_hbm.at[pl.ds(base, bpw)], iidx_v)
    hb = _TBLK // 2
    hs = hb.bit_length() - 1
    for j in range(bpw // 16):
        s = pl.ds(16 * j, 16)
        u = uidx_v[s]
        i = iidx_v[s]
        uhalf_v[s] = ((u >> (hs + 1)) << hs) | (u & (hb - 1))
        ihalf_v[s] = ((i >> (hs + 1)) << hs) | (i & (hb - 1))
        upar_v[s] = ((u >> hs) & 1) << 6
        ipar_v[s] = ((i >> hs) & 1) << 6

    for c in range(bpw // _CH):
        off = c * _CH
        cu = pltpu.async_copy(t1_hbm.at[uhalf_v.at[pl.ds(off, _CH)]],
                              urows_v, sem_u)
        ci = pltpu.async_copy(t2_hbm.at[ihalf_v.at[pl.ds(off, _CH)]],
                              irows_v, sem_i)
        cu.wait()
        ci.wait()

        def repack(j, carry):
            rows = lax.broadcasted_iota(jnp.int32, (16,), 0) + 16 * j
            pu = upar_v[pl.ds(off + 16 * j, 16)]
            pi = ipar_v[pl.ds(off + 16 * j, 16)]
            for col in range(EMB):
                cc = jnp.full((16,), col, jnp.int32)
                vu = plsc.load_gather(urows_v, [rows, pu + col])
                plsc.store_scatter(cat_v, [rows, cc], vu)
                vi = plsc.load_gather(irows_v, [rows, pi + col])
                plsc.store_scatter(cat_v, [rows, cc + EMB], vi)
            return carry

        lax.fori_loop(0, _CH // 16, repack, 0)
        pltpu.sync_copy(cat_v, x_hbm.at[pl.ds(base + off, _CH)])


def _sc_gather(user_ids, item_ids, t1, t2):
    bpw = BATCH // _NW
    mesh = plsc.VectorSubcoreMesh(core_axis_name="c", subcore_axis_name="s")
    out_type = jax.ShapeDtypeStruct((BATCH, 2 * EMB), jnp.float32)
    scratch = [
        pltpu.VMEM((bpw,), jnp.int32),
        pltpu.VMEM((bpw,), jnp.int32),
        pltpu.VMEM((bpw,), jnp.int32),
        pltpu.VMEM((bpw,), jnp.int32),
        pltpu.VMEM((bpw,), jnp.int32),
        pltpu.VMEM((bpw,), jnp.int32),
        pltpu.VMEM((_CH, 2 * EMB), jnp.float32),
        pltpu.VMEM((_CH, 2 * EMB), jnp.float32),
        pltpu.VMEM((_CH, 2 * EMB), jnp.float32),
        pltpu.SemaphoreType.DMA,
        pltpu.SemaphoreType.DMA,
    ]
    k = pl.kernel(
        functools.partial(_gather_body, bpw=bpw),
        out_type=out_type,
        mesh=mesh,
        scratch_types=scratch,
        compiler_params=pltpu.CompilerParams(
            use_tc_tiling_on_sc=False,
            needs_layout_passes=False,
        ),
    )
    return k(user_ids, item_ids, t1, t2)


def _layernorm(x, g, b, eps=1e-5):
    mu = jnp.mean(x, axis=-1, keepdims=True)
    var = jnp.mean((x - mu) ** 2, axis=-1, keepdims=True)
    return (x - mu) / jnp.sqrt(var + eps) * g + b


def _gelu(x):
    return 0.5 * x * (1.0 + lax.erf(x * (2.0 ** -0.5)))


def _mlp_body(xin, W0, b0, g0, beta0, W1, b1, g1, beta1,
              W2, b2, g2, beta2, W_out, b_out, out):
    dot = functools.partial(jnp.dot, preferred_element_type=jnp.float32)
    x = dot(xin[...], W0[...]) + b0[...]
    x = _gelu(_layernorm(x, g0[...], beta0[...]))
    x = dot(x, W1[...]) + b1[...]
    x = _gelu(_layernorm(x, g1[...], beta1[...]))
    x = dot(x, W2[...]) + b2[...]
    x = _gelu(_layernorm(x, g2[...], beta2[...]))
    out[...] = dot(x, W_out[...]) + b_out[...]


def _tc_mlp(x, W0, b0, g0, beta0, W1, b1, g1, beta1,
            W2, b2, g2, beta2, W_out, b_out):
    blk = 2048
    grid = (BATCH // blk,)

    def full_spec(a):
        return pl.BlockSpec(a.shape, lambda i: (0,) * a.ndim)

    b0r, g0r, beta0r = (a.reshape(1, -1) for a in (b0, g0, beta0))
    b1r, g1r, beta1r = (a.reshape(1, -1) for a in (b1, g1, beta1))
    b2r, g2r, beta2r = (a.reshape(1, -1) for a in (b2, g2, beta2))
    b_outr = b_out.reshape(1, -1)

    args = (x, W0, b0r, g0r, beta0r, W1, b1r, g1r, beta1r,
            W2, b2r, g2r, beta2r, W_out, b_outr)
    in_specs = ([pl.BlockSpec((blk, 2 * EMB), lambda i: (i, 0))]
                + [full_spec(a) for a in args[1:]])
    return pl.pallas_call(
        _mlp_body,
        grid=grid,
        in_specs=in_specs,
        out_specs=pl.BlockSpec((blk, 1), lambda i: (i, 0)),
        out_shape=jax.ShapeDtypeStruct((BATCH, 1), jnp.float32),
    )(*args)


def kernel(user_ids, item_ids, user_table, item_table,
           W0, b0, g0, beta0, W1, b1, g1, beta1, W2, b2, g2, beta2,
           W_out, b_out):
    t1, t2 = _detranspose(user_table.T, item_table.T)
    x = _sc_gather(user_ids.astype(jnp.int32), item_ids.astype(jnp.int32),
                   t1, t2)
    return _tc_mlp(x, W0, b0, g0, beta0, W1, b1, g1, beta1,
                   W2, b2, g2, beta2, W_out, b_out)


# speedup vs baseline: 1.9052x; 1.0086x over previous
import functools

import jax
import jax.numpy as jnp
from jax import lax
from jax.experimental import pallas as pl
from jax.experimental.pallas import tpu as pltpu
from jax.experimental.pallas import tpu_sc as plsc

BATCH = 16384
EMB = 64

_NC = 2
_NS = 16
_NW = _NC * _NS

_TBLK = 16384


def _detrans_body(t1_ref, t2_ref, o1_ref, o2_ref):
    h = _TBLK // 2
    for t_ref, o_ref in ((t1_ref, o1_ref), (t2_ref, o2_ref)):
        y = t_ref[...].T
        o_ref[:, :EMB] = y[:h]
        o_ref[:, EMB:] = y[h:]


def _detranspose(t1T, t2T):
    n = t1T.shape[1]
    grid = (pl.cdiv(n, _TBLK),)
    in_spec = pl.BlockSpec((EMB, _TBLK), lambda i: (0, i))
    out_spec = pl.BlockSpec((_TBLK // 2, 2 * EMB), lambda i: (i, 0))
    nout = (_TBLK // 2) * pl.cdiv(n, _TBLK)
    out_shape = jax.ShapeDtypeStruct((nout, 2 * EMB), jnp.float32)
    return pl.pallas_call(
        _detrans_body,
        grid=grid,
        in_specs=[in_spec, in_spec],
        out_specs=[out_spec, out_spec],
        out_shape=[out_shape, out_shape],
    )(t1T, t2T)


_CH = 128


def _gather_body(uids_hbm, iids_hbm, t1_hbm, t2_hbm, x_hbm,
                 uidx_v, iidx_v, uhalf_v, ihalf_v, upar_v, ipar_v,
                 urows_v, irows_v, cat_v, urows2_v, irows2_v, cat2_v,
                 sem_u, sem_i, sem_u2, sem_i2, bpw):
    wid = lax.axis_index("s") * _NC + lax.axis_index("c")
    base = wid * bpw
    pltpu.sync_copy(uids_hbm.at[pl.ds(base, bpw)], uidx_v)
    pltpu.sync_copy(iids_hbm.at[pl.ds(base, bpw)], iidx_v)
    hb = _TBLK // 2
    hs = hb.bit_length() - 1
    for j in range(bpw // 16):
        s = pl.ds(16 * j, 16)
        u = uidx_v[s]
        i = iidx_v[s]
        uhalf_v[s] = ((u >> (hs + 1)) << hs) | (u & (hb - 1))
        ihalf_v[s] = ((i >> (hs + 1)) << hs) | (i & (hb - 1))
        upar_v[s] = ((u >> hs) & 1) << 6
        ipar_v[s] = ((i >> hs) & 1) << 6

    nchunks = bpw // _CH

    def issue(c, bufs):
        urows, irows, _, sem_ua, sem_ia = bufs
        off = c * _CH
        cu = pltpu.async_copy(t1_hbm.at[uhalf_v.at[pl.ds(off, _CH)]],
                              urows, sem_ua)
        ci = pltpu.async_copy(t2_hbm.at[ihalf_v.at[pl.ds(off, _CH)]],
                              irows, sem_ia)
        return cu, ci

    bufs = [(urows_v, irows_v, cat_v, sem_u, sem_i),
            (urows2_v, irows2_v, cat2_v, sem_u2, sem_i2)]
    pend = issue(0, bufs[0])
    for c in range(nchunks):
        urows, irows, cat, _, _ = bufs[c % 2]
        nxt = issue(c + 1, bufs[(c + 1) % 2]) if c + 1 < nchunks else None
        pend[0].wait()
        pend[1].wait()
        off = c * _CH

        def repack(j, carry):
            rows = lax.broadcasted_iota(jnp.int32, (16,), 0) + 16 * j
            pu = upar_v[pl.ds(off + 16 * j, 16)]
            pi = ipar_v[pl.ds(off + 16 * j, 16)]
            for col in range(EMB):
                cc = jnp.full((16,), col, jnp.int32)
                vu = plsc.load_gather(urows, [rows, pu + col])
                plsc.store_scatter(cat, [rows, cc], vu)
                vi = plsc.load_gather(irows, [rows, pi + col])
                plsc.store_scatter(cat, [rows, cc + EMB], vi)
            return carry

        lax.fori_loop(0, _CH // 16, repack, 0)
        pltpu.sync_copy(cat, x_hbm.at[pl.ds(base + off, _CH)])
        pend = nxt


def _sc_gather(user_ids, item_ids, t1, t2):
    bpw = BATCH // _NW
    mesh = plsc.VectorSubcoreMesh(core_axis_name="c", subcore_axis_name="s")
    out_type = jax.ShapeDtypeStruct((BATCH, 2 * EMB), jnp.float32)
    scratch = [
        pltpu.VMEM((bpw,), jnp.int32),
        pltpu.VMEM((bpw,), jnp.int32),
        pltpu.VMEM((bpw,), jnp.int32),
        pltpu.VMEM((bpw,), jnp.int32),
        pltpu.VMEM((bpw,), jnp.int32),
        pltpu.VMEM((bpw,), jnp.int32),
        pltpu.VMEM((_CH, 2 * EMB), jnp.float32),
        pltpu.VMEM((_CH, 2 * EMB), jnp.float32),
        pltpu.VMEM((_CH, 2 * EMB), jnp.float32),
        pltpu.VMEM((_CH, 2 * EMB), jnp.float32),
        pltpu.VMEM((_CH, 2 * EMB), jnp.float32),
        pltpu.VMEM((_CH, 2 * EMB), jnp.float32),
        pltpu.SemaphoreType.DMA,
        pltpu.SemaphoreType.DMA,
        pltpu.SemaphoreType.DMA,
        pltpu.SemaphoreType.DMA,
    ]
    k = pl.kernel(
        functools.partial(_gather_body, bpw=bpw),
        out_type=out_type,
        mesh=mesh,
        scratch_types=scratch,
        compiler_params=pltpu.CompilerParams(
            use_tc_tiling_on_sc=False,
            needs_layout_passes=False,
        ),
    )
    return k(user_ids, item_ids, t1, t2)


def _layernorm(x, g, b, eps=1e-5):
    mu = jnp.mean(x, axis=-1, keepdims=True)
    var = jnp.mean((x - mu) ** 2, axis=-1, keepdims=True)
    return (x - mu) / jnp.sqrt(var + eps) * g + b


def _gelu(x):
    return 0.5 * x * (1.0 + lax.erf(x * (2.0 ** -0.5)))


def _mlp_body(xin, W0, b0, g0, beta0, W1, b1, g1, beta1,
              W2, b2, g2, beta2, W_out, b_out, out):
    dot = functools.partial(jnp.dot, preferred_element_type=jnp.float32)
    x = dot(xin[...], W0[...]) + b0[...]
    x = _gelu(_layernorm(x, g0[...], beta0[...]))
    x = dot(x, W1[...]) + b1[...]
    x = _gelu(_layernorm(x, g1[...], beta1[...]))
    x = dot(x, W2[...]) + b2[...]
    x = _gelu(_layernorm(x, g2[...], beta2[...]))
    out[...] = dot(x, W_out[...]) + b_out[...]


def _tc_mlp(x, W0, b0, g0, beta0, W1, b1, g1, beta1,
            W2, b2, g2, beta2, W_out, b_out):
    blk = 2048
    grid = (BATCH // blk,)

    def full_spec(a):
        return pl.BlockSpec(a.shape, lambda i: (0,) * a.ndim)

    b0r, g0r, beta0r = (a.reshape(1, -1) for a in (b0, g0, beta0))
    b1r, g1r, beta1r = (a.reshape(1, -1) for a in (b1, g1, beta1))
    b2r, g2r, beta2r = (a.reshape(1, -1) for a in (b2, g2, beta2))
    b_outr = b_out.reshape(1, -1)

    args = (x, W0, b0r, g0r, beta0r, W1, b1r, g1r, beta1r,
            W2, b2r, g2r, beta2r, W_out, b_outr)
    in_specs = ([pl.BlockSpec((blk, 2 * EMB), lambda i: (i, 0))]
                + [full_spec(a) for a in args[1:]])
    return pl.pallas_call(
        _mlp_body,
        grid=grid,
        in_specs=in_specs,
        out_specs=pl.BlockSpec((blk, 1), lambda i: (i, 0)),
        out_shape=jax.ShapeDtypeStruct((BATCH, 1), jnp.float32),
    )(*args)


def kernel(user_ids, item_ids, user_table, item_table,
           W0, b0, g0, beta0, W1, b1, g1, beta1, W2, b2, g2, beta2,
           W_out, b_out):
    t1, t2 = _detranspose(user_table.T, item_table.T)
    x = _sc_gather(user_ids.astype(jnp.int32), item_ids.astype(jnp.int32),
                   t1, t2)
    return _tc_mlp(x, W0, b0, g0, beta0, W1, b1, g1, beta1,
                   W2, b2, g2, beta2, W_out, b_out)
